# baseline (speedup 1.0000x reference)
import functools

import jax
import jax.numpy as jnp
from jax import lax
from jax.experimental import pallas as pl
from jax.experimental.pallas import tpu as pltpu
from jax.experimental.pallas import tpu_sc as plsc

N_NODES = 10000
NUM_EDGES = 320000
NC = 2
NS = 16
NW = NC * NS
CHUNK = 80
NBUF = 4
NIDX = 8
NCHUNKS = NIDX * (-(-NUM_EDGES // (NW * CHUNK * NIDX)))
E_PAD = NW * NCHUNKS * CHUNK
ACC_ROWS = N_NODES + 112
RPS = ACC_ROWS // NS
BN = 1000
GRID = N_NODES // BN


def _make_sc_agg(feat):
  mesh = plsc.VectorSubcoreMesh(core_axis_name="c", subcore_axis_name="s")

  @functools.partial(
      pl.kernel,
      out_type=jax.ShapeDtypeStruct((NC, ACC_ROWS, feat), jnp.float32),
      mesh=mesh,
      scratch_types=[
          [pltpu.VMEM((2, CHUNK), jnp.int32) for _ in range(NIDX)],
          [pltpu.VMEM((CHUNK, feat), jnp.float32) for _ in range(NBUF)],
          pltpu.VMEM_SHARED((ACC_ROWS, feat), jnp.float32),
          [pltpu.SemaphoreType.DMA for _ in range(NIDX)],
          [pltpu.SemaphoreType.DMA for _ in range(NBUF)],
          [pltpu.SemaphoreType.DMA for _ in range(NBUF)],
      ],
  )
  def sc_agg(g_hbm, e_hbm, zeros_hbm, out_hbm, idx, rows, acc_sh, si, sg, ss):
    cid = lax.axis_index("c")
    sid = lax.axis_index("s")
    wid = sid * NC + cid
    pltpu.sync_copy(zeros_hbm.at[pl.ds(sid * RPS, RPS)],
                    acc_sh.at[pl.ds(sid * RPS, RPS)])

    def idx_load(i, d):
      pltpu.async_copy(e_hbm.at[wid, i], idx[d], si[d])

    def idx_wait(d):
      pltpu.make_async_copy(e_hbm.at[wid, 0], idx[d], si[d]).wait()

    def gather(d, b):
      pltpu.async_copy(g_hbm.at[idx[d].at[0]], rows[b], sg[b])

    def gather_wait(d, b):
      pltpu.make_async_copy(g_hbm.at[idx[d].at[0]], rows[b], sg[b]).wait()

    def scatter_and_wait(d, b):
      pltpu.async_copy(rows[b], acc_sh.at[idx[d].at[1]], ss[b],
                       add=True).wait()

    for d in range(NIDX):
      idx_load(d, d)
    for b in range(NBUF):
      idx_wait(b)
      gather(b, b)
    plsc.subcore_barrier()

    @pl.loop(0, (NCHUNKS - NIDX) // NIDX)
    def _(k):
      i0 = k * NIDX
      for j in range(NIDX):
        b = j % NBUF
        i = i0 + j
        gather_wait(j, b)
        scatter_and_wait(j, b)
        idx_load(i + NIDX, j)
        d2 = (j + NBUF) % NIDX
        idx_wait(d2)
        gather(d2, b)

    for j in range(NIDX):
      b = j % NBUF
      gather_wait(j, b)
      scatter_and_wait(j, b)
      if j + NBUF < NIDX:
        idx_wait(j + NBUF)
        gather(j + NBUF, b)

    plsc.subcore_barrier()
    pltpu.sync_copy(acc_sh.at[pl.ds(sid * RPS, RPS)],
                    out_hbm.at[cid, pl.ds(sid * RPS, RPS)])

  return sc_agg


_sc_agg128 = _make_sc_agg(128)

_DEG_W = 16


def _make_sc_deg():
  mesh = plsc.VectorSubcoreMesh(core_axis_name="c", subcore_axis_name="s")

  @functools.partial(
      pl.kernel,
      out_type=jax.ShapeDtypeStruct((NC, ACC_ROWS, _DEG_W), jnp.float32),
      mesh=mesh,
      scratch_types=[
          pltpu.VMEM((NCHUNKS, 2, CHUNK), jnp.int32),
          pltpu.VMEM((CHUNK, _DEG_W), jnp.float32),
          pltpu.VMEM_SHARED((ACC_ROWS, _DEG_W), jnp.float32),
      ],
  )
  def sc_deg(e_hbm, zeros_hbm, out_hbm, e_v, ones_v, acc_sh):
    cid = lax.axis_index("c")
    sid = lax.axis_index("s")
    wid = sid * NC + cid
    pltpu.sync_copy(zeros_hbm.at[pl.ds(sid * RPS, RPS)],
                    acc_sh.at[pl.ds(sid * RPS, RPS)])
    pltpu.sync_copy(e_hbm.at[wid], e_v)

    @pl.loop(0, CHUNK)
    def _(r):
      ones_v[r] = jnp.ones((_DEG_W,), jnp.float32)

    plsc.subcore_barrier()

    @pl.loop(0, NCHUNKS)
    def _(i):
      pltpu.sync_copy(ones_v, acc_sh.at[e_v.at[i, 1]], add=True)

    plsc.subcore_barrier()
    pltpu.sync_copy(acc_sh.at[pl.ds(sid * RPS, RPS)],
                    out_hbm.at[cid, pl.ds(sid * RPS, RPS)])

  return sc_deg


_sc_deg = _make_sc_deg()


def _tc1_body(x_ref, w_ref, degp_ref, g_ref, dinv_ref):
  deg = degp_ref[0, :, 0:1] + degp_ref[1, :, 0:1] + 1.0
  dinv = lax.rsqrt(deg)
  g_ref[...] = dinv * jnp.dot(x_ref[...], w_ref[...],
                              preferred_element_type=jnp.float32)
  dinv_ref[...] = dinv


def _tc1(x, w1, degp):
  return pl.pallas_call(
      _tc1_body,
      grid=(GRID,),
      in_specs=[
          pl.BlockSpec((BN, 128), lambda i: (i, 0)),
          pl.BlockSpec((128, 128), lambda i: (0, 0)),
          pl.BlockSpec((NC, BN, _DEG_W), lambda i: (0, i, 0)),
      ],
      out_specs=[
          pl.BlockSpec((BN, 128), lambda i: (i, 0)),
          pl.BlockSpec((BN, 1), lambda i: (i, 0)),
      ],
      out_shape=[
          jax.ShapeDtypeStruct((N_NODES, 128), jnp.float32),
          jax.ShapeDtypeStruct((N_NODES, 1), jnp.float32),
      ],
  )(x, w1, degp)


def _tc_mid_body(p_ref, g_ref, dinv_ref, b_ref, w_ref, out_ref):
  dinv = dinv_ref[...]
  h = p_ref[0] + p_ref[1] + g_ref[...]
  h = jnp.maximum(dinv * h + b_ref[...], 0.0)
  out_ref[...] = dinv * jnp.dot(h, w_ref[...],
                                preferred_element_type=jnp.float32)


def _tc_mid(p, g, dinv, b, w, fout):
  return pl.pallas_call(
      _tc_mid_body,
      grid=(GRID,),
      in_specs=[
          pl.BlockSpec((NC, BN, 128), lambda i: (0, i, 0)),
          pl.BlockSpec((BN, 128), lambda i: (i, 0)),
          pl.BlockSpec((BN, 1), lambda i: (i, 0)),
          pl.BlockSpec((1, 128), lambda i: (0, 0)),
          pl.BlockSpec((128, fout), lambda i: (0, 0)),
      ],
      out_specs=pl.BlockSpec((BN, fout), lambda i: (i, 0)),
      out_shape=jax.ShapeDtypeStruct((N_NODES, fout), jnp.float32),
  )(p, g, dinv, b, w)


def _tc_q3_body(p_ref, g_ref, dinv_ref, b_ref, q_ref):
  dinv = dinv_ref[...]
  h = p_ref[0] + p_ref[1] + g_ref[...]
  q_ref[...] = dinv * jnp.maximum(dinv * h + b_ref[...], 0.0)


def _tc_q3(p, g, dinv, b):
  return pl.pallas_call(
      _tc_q3_body,
      grid=(GRID,),
      in_specs=[
          pl.BlockSpec((NC, BN, 128), lambda i: (0, i, 0)),
          pl.BlockSpec((BN, 128), lambda i: (i, 0)),
          pl.BlockSpec((BN, 1), lambda i: (i, 0)),
          pl.BlockSpec((1, 128), lambda i: (0, 0)),
      ],
      out_specs=pl.BlockSpec((BN, 128), lambda i: (i, 0)),
      out_shape=jax.ShapeDtypeStruct((N_NODES, 128), jnp.float32),
  )(p, g, dinv, b)


def _tc_final_body(p_ref, q_ref, dinv_ref, b_ref, w_ref, out_ref):
  h = dinv_ref[...] * (p_ref[0] + p_ref[1] + q_ref[...])
  out_ref[...] = jnp.dot(h, w_ref[...],
                         preferred_element_type=jnp.float32) + b_ref[...]


def _tc_final(p, q, dinv, b, w):
  return pl.pallas_call(
      _tc_final_body,
      grid=(GRID,),
      in_specs=[
          pl.BlockSpec((NC, BN, 128), lambda i: (0, i, 0)),
          pl.BlockSpec((BN, 128), lambda i: (i, 0)),
          pl.BlockSpec((BN, 1), lambda i: (i, 0)),
          pl.BlockSpec((1, 64), lambda i: (0, 0)),
          pl.BlockSpec((128, 64), lambda i: (0, 0)),
      ],
      out_specs=pl.BlockSpec((BN, 64), lambda i: (i, 0)),
      out_shape=jax.ShapeDtypeStruct((N_NODES, 64), jnp.float32),
  )(p, q, dinv, b, w)


def kernel(x, edge_index, W1, b1, W2, b2, W3, b3):
  pad = E_PAD - NUM_EDGES
  src = jnp.concatenate(
      [edge_index[0], jnp.zeros((pad,), jnp.int32)]).reshape(NW, NCHUNKS, CHUNK)
  dst = jnp.concatenate(
      [edge_index[1], jnp.full((pad,), N_NODES, jnp.int32)]
  ).reshape(NW, NCHUNKS, CHUNK)
  e = jnp.stack([src, dst], axis=2)

  zeros_deg = jnp.zeros((ACC_ROWS, _DEG_W), jnp.float32)
  zeros128 = jnp.zeros((ACC_ROWS, 128), jnp.float32)

  degp = _sc_deg(e, zeros_deg)
  g1, dinv = _tc1(x, W1, degp)
  p1 = _sc_agg128(g1, e, zeros128)
  g2 = _tc_mid(p1, g1, dinv, b1.reshape(1, -1), W2, 128)
  p2 = _sc_agg128(g2, e, zeros128)
  q3 = _tc_q3(p2, g2, dinv, b2.reshape(1, -1))
  p3 = _sc_agg128(q3, e, zeros128)
  return _tc_final(p3, q3, dinv, b3.reshape(1, -1), W3)


# speedup vs baseline: 1.4864x; 1.4864x over previous
import functools

import jax
import jax.numpy as jnp
from jax import lax
from jax.experimental import pallas as pl
from jax.experimental.pallas import tpu as pltpu
from jax.experimental.pallas import tpu_sc as plsc

N_NODES = 10000
NUM_EDGES = 320000
NC = 2
NS = 16
NW = NC * NS
CHUNK = 128
NCHUNKS = -(-NUM_EDGES // (NW * CHUNK))
E_PAD = NW * NCHUNKS * CHUNK
ACC_ROWS = N_NODES + 112
RPS = ACC_ROWS // NS
BN = 1000
GRID = N_NODES // BN


def _make_sc_agg(feat):
  mesh = plsc.VectorSubcoreMesh(core_axis_name="c", subcore_axis_name="s")

  @functools.partial(
      pl.kernel,
      out_type=jax.ShapeDtypeStruct((NC, ACC_ROWS, feat), jnp.float32),
      mesh=mesh,
      scratch_types=[
          pltpu.VMEM((NCHUNKS, CHUNK), jnp.int32),
          pltpu.VMEM((NCHUNKS, CHUNK), jnp.int32),
          pltpu.VMEM((CHUNK, feat), jnp.float32),
          pltpu.VMEM_SHARED((ACC_ROWS, feat), jnp.float32),
          pltpu.SemaphoreType.DMA,
      ],
  )
  def sc_agg(g_hbm, src_hbm, dst_hbm, zeros_hbm, out_hbm,
             src_v, dst_v, rows_v, acc_sh, sem):
    cid = lax.axis_index("c")
    sid = lax.axis_index("s")
    wid = sid * NC + cid
    pltpu.sync_copy(zeros_hbm.at[pl.ds(sid * RPS, RPS)],
                    acc_sh.at[pl.ds(sid * RPS, RPS)])
    pltpu.sync_copy(src_hbm.at[wid], src_v)
    pltpu.sync_copy(dst_hbm.at[wid], dst_v)
    plsc.subcore_barrier()

    @pl.loop(0, NCHUNKS)
    def _(i):
      pltpu.async_copy(g_hbm.at[src_v.at[i]], rows_v, sem).wait()
      pltpu.sync_copy(rows_v, acc_sh.at[dst_v.at[i]], add=True)

    plsc.subcore_barrier()
    pltpu.sync_copy(acc_sh.at[pl.ds(sid * RPS, RPS)],
                    out_hbm.at[cid, pl.ds(sid * RPS, RPS)])

  return sc_agg


_sc_agg128 = _make_sc_agg(128)

_DEG_W = 16


def _make_sc_deg():
  mesh = plsc.VectorSubcoreMesh(core_axis_name="c", subcore_axis_name="s")

  @functools.partial(
      pl.kernel,
      out_type=jax.ShapeDtypeStruct((NC, ACC_ROWS, _DEG_W), jnp.float32),
      mesh=mesh,
      scratch_types=[
          pltpu.VMEM((NCHUNKS, CHUNK), jnp.int32),
          pltpu.VMEM((CHUNK, _DEG_W), jnp.float32),
          pltpu.VMEM_SHARED((ACC_ROWS, _DEG_W), jnp.float32),
      ],
  )
  def sc_deg(dst_hbm, zeros_hbm, out_hbm, dst_v, ones_v, acc_sh):
    cid = lax.axis_index("c")
    sid = lax.axis_index("s")
    wid = sid * NC + cid
    pltpu.sync_copy(zeros_hbm.at[pl.ds(sid * RPS, RPS)],
                    acc_sh.at[pl.ds(sid * RPS, RPS)])
    pltpu.sync_copy(dst_hbm.at[wid], dst_v)

    @pl.loop(0, CHUNK)
    def _(r):
      ones_v[r] = jnp.ones((_DEG_W,), jnp.float32)

    plsc.subcore_barrier()

    @pl.loop(0, NCHUNKS)
    def _(i):
      pltpu.sync_copy(ones_v, acc_sh.at[dst_v.at[i]], add=True)

    plsc.subcore_barrier()
    pltpu.sync_copy(acc_sh.at[pl.ds(sid * RPS, RPS)],
                    out_hbm.at[cid, pl.ds(sid * RPS, RPS)])

  return sc_deg


_sc_deg = _make_sc_deg()


def _tc1_body(x_ref, w_ref, degp_ref, g_ref, dinv_ref):
  deg = degp_ref[0, :, 0:1] + degp_ref[1, :, 0:1] + 1.0
  dinv = lax.rsqrt(deg)
  g_ref[...] = dinv * jnp.dot(x_ref[...], w_ref[...],
                              preferred_element_type=jnp.float32)
  dinv_ref[...] = dinv


def _tc1(x, w1, degp):
  return pl.pallas_call(
      _tc1_body,
      grid=(GRID,),
      in_specs=[
          pl.BlockSpec((BN, 128), lambda i: (i, 0)),
          pl.BlockSpec((128, 128), lambda i: (0, 0)),
          pl.BlockSpec((NC, BN, _DEG_W), lambda i: (0, i, 0)),
      ],
      out_specs=[
          pl.BlockSpec((BN, 128), lambda i: (i, 0)),
          pl.BlockSpec((BN, 1), lambda i: (i, 0)),
      ],
      out_shape=[
          jax.ShapeDtypeStruct((N_NODES, 128), jnp.float32),
          jax.ShapeDtypeStruct((N_NODES, 1), jnp.float32),
      ],
  )(x, w1, degp)


def _tc_mid_body(p_ref, g_ref, dinv_ref, b_ref, w_ref, out_ref):
  dinv = dinv_ref[...]
  h = p_ref[0] + p_ref[1] + g_ref[...]
  h = jnp.maximum(dinv * h + b_ref[...], 0.0)
  out_ref[...] = dinv * jnp.dot(h, w_ref[...],
                                preferred_element_type=jnp.float32)


def _tc_mid(p, g, dinv, b, w, fout):
  return pl.pallas_call(
      _tc_mid_body,
      grid=(GRID,),
      in_specs=[
          pl.BlockSpec((NC, BN, 128), lambda i: (0, i, 0)),
          pl.BlockSpec((BN, 128), lambda i: (i, 0)),
          pl.BlockSpec((BN, 1), lambda i: (i, 0)),
          pl.BlockSpec((1, 128), lambda i: (0, 0)),
          pl.BlockSpec((128, fout), lambda i: (0, 0)),
      ],
      out_specs=pl.BlockSpec((BN, fout), lambda i: (i, 0)),
      out_shape=jax.ShapeDtypeStruct((N_NODES, fout), jnp.float32),
  )(p, g, dinv, b, w)


def _tc_q3_body(p_ref, g_ref, dinv_ref, b_ref, q_ref):
  dinv = dinv_ref[...]
  h = p_ref[0] + p_ref[1] + g_ref[...]
  q_ref[...] = dinv * jnp.maximum(dinv * h + b_ref[...], 0.0)


def _tc_q3(p, g, dinv, b):
  return pl.pallas_call(
      _tc_q3_body,
      grid=(GRID,),
      in_specs=[
          pl.BlockSpec((NC, BN, 128), lambda i: (0, i, 0)),
          pl.BlockSpec((BN, 128), lambda i: (i, 0)),
          pl.BlockSpec((BN, 1), lambda i: (i, 0)),
          pl.BlockSpec((1, 128), lambda i: (0, 0)),
      ],
      out_specs=pl.BlockSpec((BN, 128), lambda i: (i, 0)),
      out_shape=jax.ShapeDtypeStruct((N_NODES, 128), jnp.float32),
  )(p, g, dinv, b)


def _tc_final_body(p_ref, q_ref, dinv_ref, b_ref, w_ref, out_ref):
  h = dinv_ref[...] * (p_ref[0] + p_ref[1] + q_ref[...])
  out_ref[...] = jnp.dot(h, w_ref[...],
                         preferred_element_type=jnp.float32) + b_ref[...]


def _tc_final(p, q, dinv, b, w):
  return pl.pallas_call(
      _tc_final_body,
      grid=(GRID,),
      in_specs=[
          pl.BlockSpec((NC, BN, 128), lambda i: (0, i, 0)),
          pl.BlockSpec((BN, 128), lambda i: (i, 0)),
          pl.BlockSpec((BN, 1), lambda i: (i, 0)),
          pl.BlockSpec((1, 64), lambda i: (0, 0)),
          pl.BlockSpec((128, 64), lambda i: (0, 0)),
      ],
      out_specs=pl.BlockSpec((BN, 64), lambda i: (i, 0)),
      out_shape=jax.ShapeDtypeStruct((N_NODES, 64), jnp.float32),
  )(p, q, dinv, b, w)


def kernel(x, edge_index, W1, b1, W2, b2, W3, b3):
  pad = E_PAD - NUM_EDGES
  src = jnp.concatenate(
      [edge_index[0], jnp.zeros((pad,), jnp.int32)]).reshape(NW, NCHUNKS, CHUNK)
  dst = jnp.concatenate(
      [edge_index[1], jnp.full((pad,), N_NODES, jnp.int32)]
  ).reshape(NW, NCHUNKS, CHUNK)

  zeros_deg = jnp.zeros((ACC_ROWS, _DEG_W), jnp.float32)
  zeros128 = jnp.zeros((ACC_ROWS, 128), jnp.float32)

  degp = _sc_deg(dst, zeros_deg)
  g1, dinv = _tc1(x, W1, degp)
  p1 = _sc_agg128(g1, src, dst, zeros128)
  g2 = _tc_mid(p1, g1, dinv, b1.reshape(1, -1), W2, 128)
  p2 = _sc_agg128(g2, src, dst, zeros128)
  q3 = _tc_q3(p2, g2, dinv, b2.reshape(1, -1))
  p3 = _sc_agg128(q3, src, dst, zeros128)
  return _tc_final(p3, q3, dinv, b3.reshape(1, -1), W3)
